# baseline (speedup 1.0000x reference)
import functools
import math

import jax
import jax.numpy as jnp
from jax import lax
from jax.experimental import pallas as pl
from jax.experimental.pallas import tpu as pltpu
from jax.experimental.pallas import tpu_sc as plsc

N = 50000
TOTAL = 32768
NPATHS = 8192
T = 64
F = 33
FP = 48
BN = 1024
NPAD = 50176
BP = 512
CHK = 128
NW = 32
PPW = NPATHS // NW
HI = jax.lax.Precision.HIGHEST
LO = jax.lax.Precision.DEFAULT


def _softplus(x):
    return jnp.maximum(x, 0.0) + jnp.log1p(jnp.exp(-jnp.abs(x)))


def _stage_a1(p_ref, cc_ref, ss_ref, tab_ref):
    p = p_ref[:, :]
    a = _softplus(p[:, 0:1]) + 0.5
    b = _softplus(p[:, 1:2]) + 0.1
    ti = lax.broadcasted_iota(jnp.int32, (1, T), 1).astype(jnp.float32)
    t = ti + 1.0
    u = jnp.exp(a * jnp.log(t) - b * t)
    inv = 1.0 / (jnp.sum(u, axis=-1, keepdims=True) + 1e-12)
    re = jnp.dot(u, cc_ref[:, :], precision=HI) * inv
    im = jnp.dot(u, ss_ref[:, :], precision=HI) * inv
    tab_ref[:, :] = jnp.concatenate(
        [re, im, jnp.zeros((re.shape[0], 128 - 2 * FP), jnp.float32)], axis=1)


def _stage_a2(p_ref, diag_ref):
    p = p_ref[:, :]
    a = _softplus(p[0:1, :]) + 0.5
    b = _softplus(p[1:2, :]) + 0.1
    ti = lax.broadcasted_iota(jnp.int32, (T, 1), 0).astype(jnp.float32)
    tf = jnp.float32(T) - ti
    uf = jnp.exp(a * jnp.log(tf) - b * tf)
    s = jnp.sum(uf, axis=0, keepdims=True)
    diag_ref[:, :] = uf * (1.0 / (s * (1.0 + 1e-8) + 1e-20))


def _stage_b(tab_hbm, nodes_hbm, cs_hbm, cs1_hbm,
             osum_hbm, src_hbm, dst_hbm,
             nodes_v, cs_v, cs1_v, gbuf_v, acc_v, bval_v, sem):
    wid = lax.axis_index("s") * 2 + lax.axis_index("c")
    p0 = pl.multiple_of(wid * PPW, PPW)
    pltpu.sync_copy(nodes_hbm, nodes_v)
    pltpu.sync_copy(cs_hbm.at[pl.ds(p0, PPW + 16)], cs_v)
    pltpu.sync_copy(cs1_hbm.at[pl.ds(p0, PPW + 16)], cs1_v)

    for k in range(PPW // 16):
        c0 = cs_v[pl.ds(k * 16, 16)]
        c1 = cs1_v[pl.ds(k * 16, 16)]
        s_pos = jnp.clip(c0, 0, TOTAL - 1)
        d_pos = jnp.clip(c1 - 1, 0, TOTAL - 1)
        bval_v[pl.ds(k * 16, 16)] = plsc.load_gather(nodes_v, [s_pos])
        bval_v[pl.ds(PPW + k * 16, 16)] = plsc.load_gather(nodes_v, [d_pos])
    pltpu.sync_copy(bval_v.at[pl.ds(0, PPW)], src_hbm.at[pl.ds(p0, PPW)])
    pltpu.sync_copy(bval_v.at[pl.ds(PPW, PPW)], dst_hbm.at[pl.ds(p0, PPW)])

    one = jnp.ones((16,), jnp.float32)
    zero = jnp.zeros((16,), jnp.float32)

    def zbody(r, _):
        for c in range(FP // 16):
            acc_v[r, pl.ds(c * 16, 16)] = one
        for c in range(FP // 16, 8):
            acc_v[r, pl.ds(c * 16, 16)] = zero
        return 0

    lax.fori_loop(0, PPW, zbody, 0)

    c_lo = cs_v[pl.ds(0, 16)][0]
    c_hi = cs_v[pl.ds(PPW, 16)][0]
    s0 = (c_lo // 8) * 8
    nch = (c_hi - s0 + CHK - 1) // CHK

    def _advance(rp, i):
        v = cs_v[pl.ds(rp + 1, 16)]
        ind = jnp.minimum(jnp.maximum(i - v + 1, 0), 1)
        return jnp.minimum(rp + jnp.sum(ind), PPW - 1)

    def do_chunk(ci, rp):
        s = s0 + ci * CHK
        pltpu.async_copy(tab_hbm.at[nodes_v.at[pl.ds(s, CHK)]], gbuf_v,
                         sem).wait()

        def item(j, rp):
            i = s + j
            rp1 = _advance(rp, i)

            def catchup(r):
                def step(_, r):
                    return _advance(r, i)
                return lax.fori_loop(0, PPW // 16 - 1, step, r)

            rp = lax.cond(rp1 - rp == 16, catchup, lambda r: r, rp1)

            @pl.when(jnp.logical_and(i >= c_lo, i < c_hi))
            def _():
                for c in range(FP // 16):
                    slr = pl.ds(c * 16, 16)
                    sli = pl.ds(FP + c * 16, 16)
                    ar = acc_v[rp, slr]
                    ai = acc_v[rp, sli]
                    br = gbuf_v[j, slr]
                    bi = gbuf_v[j, sli]
                    acc_v[rp, slr] = ar * br - ai * bi
                    acc_v[rp, sli] = ar * bi + ai * br

            return rp

        return lax.fori_loop(0, CHK, item, rp)

    def chunk(ci, rp):
        return lax.cond(ci < nch, do_chunk, lambda _, r: r, ci, rp)

    NCH_MAX = (TOTAL + 7 + CHK - 1) // CHK + 1
    lax.fori_loop(0, NCH_MAX, chunk, jnp.int32(0))

    pltpu.sync_copy(acc_v, osum_hbm.at[pl.ds(p0, PPW)])


def _stage_c(sum_ref, big_ref, ic_ref, is_ref, o_ref):
    del big_ref
    x = sum_ref[:, :]
    fr = x[:, 0:FP]
    fi = x[:, FP:2 * FP]
    dn = (((1,), (1,)), ((), ()))
    y = (lax.dot_general(ic_ref[:, :], fr, dn, precision=LO)
         + lax.dot_general(is_ref[:, :], fi, dn, precision=LO))
    y = jnp.maximum(y, 0.0)
    o_ref[:, :] = y / (jnp.sum(y, axis=0, keepdims=True) + 1e-8)


@jax.jit
def kernel(params, path_nodes, path_cumsum):
    f32 = jnp.float32
    tt = jnp.arange(T, dtype=f32)
    kk = jnp.arange(FP, dtype=f32)
    kmask = (kk < F).astype(f32)
    ang = 2.0 * math.pi * jnp.outer(tt, kk) / T
    cc = jnp.cos(ang) * kmask
    ss = -jnp.sin(ang) * kmask
    w = jnp.where((kk == 0) | (kk == F - 1), 1.0, 2.0) * kmask
    tflip = (T - 1) - tt
    ang2 = 2.0 * math.pi * jnp.outer(tflip, kk) / T
    ic = jnp.cos(ang2) * w / T
    isn = -jnp.sin(ang2) * w / T

    params_t = jnp.pad(params.T, ((0, 0), (0, NPAD - N)))

    grid_a = NPAD // BN
    tab = pl.pallas_call(
        _stage_a1,
        grid=(grid_a,),
        in_specs=[
            pl.BlockSpec((BN, 2), lambda i: (i, 0)),
            pl.BlockSpec((T, FP), lambda i: (0, 0)),
            pl.BlockSpec((T, FP), lambda i: (0, 0)),
        ],
        out_specs=pl.BlockSpec((BN, 128), lambda i: (i, 0)),
        out_shape=jax.ShapeDtypeStruct((NPAD, 128), f32),
    )(params, cc, ss)

    NOUT = NPATHS + N
    big = pl.pallas_call(
        _stage_a2,
        grid=(grid_a,),
        in_specs=[
            pl.BlockSpec((2, BN), lambda i: (0, i)),
        ],
        out_specs=pl.BlockSpec((T, BN), lambda i: (0, i + NPATHS // BN)),
        out_shape=jax.ShapeDtypeStruct((T, NOUT), f32),
    )(params_t)

    nodes_pad = jnp.zeros((TOTAL + CHK + 8,), jnp.int32).at[:TOTAL].set(
        path_nodes.astype(jnp.int32))
    cs_pad = jnp.full((NPATHS + 16,), TOTAL, jnp.int32).at[:NPATHS + 1].set(
        path_cumsum.astype(jnp.int32))
    cs1_pad = jnp.full((NPATHS + 16,), TOTAL, jnp.int32).at[:NPATHS].set(
        path_cumsum[1:].astype(jnp.int32))

    mesh = plsc.VectorSubcoreMesh(core_axis_name="c", subcore_axis_name="s")
    sums, src, dst = pl.kernel(
        _stage_b,
        out_type=[
            jax.ShapeDtypeStruct((NPATHS, 128), f32),
            jax.ShapeDtypeStruct((NPATHS,), jnp.int32),
            jax.ShapeDtypeStruct((NPATHS,), jnp.int32),
        ],
        mesh=mesh,
        scratch_types=[
            pltpu.VMEM((TOTAL + CHK + 8,), jnp.int32),
            pltpu.VMEM((PPW + 16,), jnp.int32),
            pltpu.VMEM((PPW + 16,), jnp.int32),
            pltpu.VMEM((CHK, 128), f32),
            pltpu.VMEM((PPW, 128), f32),
            pltpu.VMEM((2 * PPW,), jnp.int32),
            pltpu.SemaphoreType.DMA,
        ],
        compiler_params=pltpu.CompilerParams(needs_layout_passes=False),
    )(tab, nodes_pad, cs_pad, cs1_pad)

    grid_c = NPATHS // BP
    out_t = pl.pallas_call(
        _stage_c,
        grid=(grid_c,),
        in_specs=[
            pl.BlockSpec((BP, 128), lambda i: (i, 0)),
            pl.BlockSpec(memory_space=pl.ANY),
            pl.BlockSpec((T, FP), lambda i: (0, 0)),
            pl.BlockSpec((T, FP), lambda i: (0, 0)),
        ],
        out_specs=pl.BlockSpec((T, BP), lambda i: (0, i)),
        out_shape=jax.ShapeDtypeStruct((T, NOUT), f32),
        input_output_aliases={1: 0},
    )(sums, big, ic, isn)
    irfs_agg = out_t.T

    diag_idx = jnp.arange(N, dtype=src.dtype)
    coords = jnp.stack([jnp.concatenate([src, diag_idx]),
                        jnp.concatenate([dst, diag_idx])], axis=0)
    return (coords, irfs_agg)


# speedup vs baseline: 1.2908x; 1.2908x over previous
import functools
import math

import jax
import jax.numpy as jnp
from jax import lax
from jax.experimental import pallas as pl
from jax.experimental.pallas import tpu as pltpu
from jax.experimental.pallas import tpu_sc as plsc

N = 50000
TOTAL = 32768
NPATHS = 8192
T = 64
F = 33
FP = 48
BN = 1024
NPAD = 50176
BP = 512
CHK = 128
NW = 32
PPW = NPATHS // NW
HI = jax.lax.Precision.HIGHEST
LO = jax.lax.Precision.DEFAULT


def _softplus(x):
    return jnp.maximum(x, 0.0) + jnp.log1p(jnp.exp(-jnp.abs(x)))


def _stage_a1(p_ref, cc_ref, ss_ref, tab_ref):
    p = p_ref[:, :]
    a = _softplus(p[:, 0:1]) + 0.5
    b = _softplus(p[:, 1:2]) + 0.1
    ti = lax.broadcasted_iota(jnp.int32, (1, T), 1).astype(jnp.float32)
    t = ti + 1.0
    u = jnp.exp(a * jnp.log(t) - b * t)
    inv = 1.0 / (jnp.sum(u, axis=-1, keepdims=True) + 1e-12)
    re = jnp.dot(u, cc_ref[:, :], precision=HI) * inv
    im = jnp.dot(u, ss_ref[:, :], precision=HI) * inv
    tab_ref[:, :] = jnp.concatenate(
        [re, im, jnp.zeros((re.shape[0], 128 - 2 * FP), jnp.float32)], axis=1)


def _stage_a2(p_ref, diag_ref):
    p = p_ref[:, :]
    a = _softplus(p[0:1, :]) + 0.5
    b = _softplus(p[1:2, :]) + 0.1
    ti = lax.broadcasted_iota(jnp.int32, (T, 1), 0).astype(jnp.float32)
    tf = jnp.float32(T) - ti
    uf = jnp.exp(a * jnp.log(tf) - b * tf)
    s = jnp.sum(uf, axis=0, keepdims=True)
    diag_ref[:, :] = uf * (1.0 / (s * (1.0 + 1e-8) + 1e-20))


def _stage_b(tab_hbm, nodes_hbm, cs_hbm, cs1_hbm,
             osum_hbm, src_hbm, dst_hbm,
             nodes_v, cs_v, cs1_v, gbuf0_v, gbuf1_v, acc_v, bval_v,
             sem, sem0, sem1):
    wid = lax.axis_index("s") * 2 + lax.axis_index("c")
    p0 = pl.multiple_of(wid * PPW, PPW)
    pltpu.sync_copy(nodes_hbm, nodes_v)
    pltpu.sync_copy(cs_hbm.at[pl.ds(p0, PPW + 16)], cs_v)
    pltpu.sync_copy(cs1_hbm.at[pl.ds(p0, PPW + 16)], cs1_v)

    for k in range(PPW // 16):
        c0 = cs_v[pl.ds(k * 16, 16)]
        c1 = cs1_v[pl.ds(k * 16, 16)]
        s_pos = jnp.clip(c0, 0, TOTAL - 1)
        d_pos = jnp.clip(c1 - 1, 0, TOTAL - 1)
        bval_v[pl.ds(k * 16, 16)] = plsc.load_gather(nodes_v, [s_pos])
        bval_v[pl.ds(PPW + k * 16, 16)] = plsc.load_gather(nodes_v, [d_pos])
    pltpu.sync_copy(bval_v.at[pl.ds(0, PPW)], src_hbm.at[pl.ds(p0, PPW)])
    pltpu.sync_copy(bval_v.at[pl.ds(PPW, PPW)], dst_hbm.at[pl.ds(p0, PPW)])

    one = jnp.ones((16,), jnp.float32)
    zero = jnp.zeros((16,), jnp.float32)

    def zbody(r, _):
        for c in range(FP // 16):
            acc_v[r, pl.ds(c * 16, 16)] = one
        for c in range(FP // 16, 8):
            acc_v[r, pl.ds(c * 16, 16)] = zero
        return 0

    lax.fori_loop(0, PPW, zbody, 0)

    c_lo = cs_v[pl.ds(0, 16)][0]
    c_hi = cs_v[pl.ds(PPW, 16)][0]
    s0 = (c_lo // 8) * 8
    nch = (c_hi - s0 + CHK - 1) // CHK

    gbufs = (gbuf0_v, gbuf1_v)
    sems = (sem0, sem1)

    def _gather(ci, b):
        return pltpu.make_async_copy(
            tab_hbm.at[nodes_v.at[pl.ds(s0 + ci * CHK, CHK)]], gbufs[b],
            sems[b])

    def _advance(rp, i):
        v = cs_v[pl.ds(rp + 1, 16)]
        ind = jnp.minimum(jnp.maximum(i - v + 1, 0), 1)
        return jnp.minimum(rp + jnp.sum(ind), PPW - 1)

    def _flush(st, i):
        rp = st[0]
        for c in range(3):
            acc_v[rp, pl.ds(c * 16, 16)] = st[2 + c]
            acc_v[rp, pl.ds(FP + c * 16, 16)] = st[5 + c]
        rp1 = _advance(rp, i)

        def catchup(r):
            return lax.fori_loop(0, PPW // 16 - 1,
                                 lambda _, rr: _advance(rr, i), r)

        rp2 = lax.cond(rp1 - rp == 16, catchup, lambda r: r, rp1)
        nxt = cs_v[pl.ds(rp2 + 1, 16)][0]
        one = jnp.ones((16,), jnp.float32)
        zero = jnp.zeros((16,), jnp.float32)
        return (rp2, nxt, one, one, one, zero, zero, zero)

    def _make_loop(buf, guarded, s_base):
        def item(j, st):
            i = s_base + j
            fl = st[1] <= i
            if guarded:
                fl = jnp.logical_and(fl, i <= c_hi)
            st = lax.cond(fl, lambda s: _flush(s, i), lambda s: s, st)
            rp, nxt, r0, r1, r2, q0, q1, q2 = st
            b0 = buf[j, pl.ds(0, 16)]
            b1 = buf[j, pl.ds(16, 16)]
            b2 = buf[j, pl.ds(32, 16)]
            d0 = buf[j, pl.ds(FP, 16)]
            d1 = buf[j, pl.ds(FP + 16, 16)]
            d2 = buf[j, pl.ds(FP + 32, 16)]
            n0 = r0 * b0 - q0 * d0
            n1 = r1 * b1 - q1 * d1
            n2 = r2 * b2 - q2 * d2
            m0 = r0 * d0 + q0 * b0
            m1 = r1 * d1 + q1 * b1
            m2 = r2 * d2 + q2 * b2
            if guarded:
                ok = jnp.logical_and(i >= c_lo, i < c_hi)
                n0 = jnp.where(ok, n0, r0)
                n1 = jnp.where(ok, n1, r1)
                n2 = jnp.where(ok, n2, r2)
                m0 = jnp.where(ok, m0, q0)
                m1 = jnp.where(ok, m1, q1)
                m2 = jnp.where(ok, m2, q2)
            return (rp, nxt, n0, n1, n2, m0, m1, m2)

        return item

    def _proc(ci, st, b):
        _gather(ci, b).wait()
        s_base = s0 + ci * CHK
        edge = jnp.logical_or(ci == 0, ci == nch - 1)
        st = lax.cond(
            edge,
            lambda s: lax.fori_loop(0, CHK, _make_loop(gbufs[b], True,
                                                       s_base), s),
            lambda s: lax.fori_loop(0, CHK, _make_loop(gbufs[b], False,
                                                       s_base), s),
            st)

        @pl.when(ci + 2 < nch)
        def _():
            _gather(ci + 2, b).start()

        return st

    @pl.when(0 < nch)
    def _():
        _gather(0, 0).start()

    @pl.when(1 < nch)
    def _():
        _gather(1, 1).start()

    one0 = jnp.ones((16,), jnp.float32)
    zero0 = jnp.zeros((16,), jnp.float32)
    st0 = (jnp.int32(0), cs_v[pl.ds(1, 16)][0],
           one0, one0, one0, zero0, zero0, zero0)

    def chunk_pair(cc, st):
        ci = cc * 2
        st = lax.cond(ci < nch, lambda s: _proc(ci, s, 0), lambda s: s, st)
        st = lax.cond(ci + 1 < nch,
                      lambda s: _proc(ci + 1, s, 1), lambda s: s, st)
        return st

    NCH_MAX = (TOTAL + 7 + CHK - 1) // CHK + 1
    stf = lax.fori_loop(0, (NCH_MAX + 1) // 2, chunk_pair, st0)

    @pl.when(s0 + nch * CHK == c_hi)
    def _():
        rp = stf[0]
        for c in range(3):
            acc_v[rp, pl.ds(c * 16, 16)] = stf[2 + c]
            acc_v[rp, pl.ds(FP + c * 16, 16)] = stf[5 + c]

    pltpu.sync_copy(acc_v, osum_hbm.at[pl.ds(p0, PPW)])


def _stage_c(sum_ref, big_ref, ic_ref, is_ref, o_ref):
    del big_ref
    x = sum_ref[:, :]
    fr = x[:, 0:FP]
    fi = x[:, FP:2 * FP]
    dn = (((1,), (1,)), ((), ()))
    y = (lax.dot_general(ic_ref[:, :], fr, dn, precision=LO)
         + lax.dot_general(is_ref[:, :], fi, dn, precision=LO))
    y = jnp.maximum(y, 0.0)
    o_ref[:, :] = y / (jnp.sum(y, axis=0, keepdims=True) + 1e-8)


@jax.jit
def kernel(params, path_nodes, path_cumsum):
    f32 = jnp.float32
    tt = jnp.arange(T, dtype=f32)
    kk = jnp.arange(FP, dtype=f32)
    kmask = (kk < F).astype(f32)
    ang = 2.0 * math.pi * jnp.outer(tt, kk) / T
    cc = jnp.cos(ang) * kmask
    ss = -jnp.sin(ang) * kmask
    w = jnp.where((kk == 0) | (kk == F - 1), 1.0, 2.0) * kmask
    tflip = (T - 1) - tt
    ang2 = 2.0 * math.pi * jnp.outer(tflip, kk) / T
    ic = jnp.cos(ang2) * w / T
    isn = -jnp.sin(ang2) * w / T

    params_t = jnp.pad(params.T, ((0, 0), (0, NPAD - N)))

    grid_a = NPAD // BN
    tab = pl.pallas_call(
        _stage_a1,
        grid=(grid_a,),
        in_specs=[
            pl.BlockSpec((BN, 2), lambda i: (i, 0)),
            pl.BlockSpec((T, FP), lambda i: (0, 0)),
            pl.BlockSpec((T, FP), lambda i: (0, 0)),
        ],
        out_specs=pl.BlockSpec((BN, 128), lambda i: (i, 0)),
        out_shape=jax.ShapeDtypeStruct((NPAD, 128), f32),
    )(params, cc, ss)

    NOUT = NPATHS + N
    big = pl.pallas_call(
        _stage_a2,
        grid=(grid_a,),
        in_specs=[
            pl.BlockSpec((2, BN), lambda i: (0, i)),
        ],
        out_specs=pl.BlockSpec((T, BN), lambda i: (0, i + NPATHS // BN)),
        out_shape=jax.ShapeDtypeStruct((T, NOUT), f32),
    )(params_t)

    nodes_pad = jnp.zeros((TOTAL + CHK + 8,), jnp.int32).at[:TOTAL].set(
        path_nodes.astype(jnp.int32))
    cs_pad = jnp.full((NPATHS + 16,), TOTAL, jnp.int32).at[:NPATHS + 1].set(
        path_cumsum.astype(jnp.int32))
    cs1_pad = jnp.full((NPATHS + 16,), TOTAL, jnp.int32).at[:NPATHS].set(
        path_cumsum[1:].astype(jnp.int32))

    mesh = plsc.VectorSubcoreMesh(core_axis_name="c", subcore_axis_name="s")
    sums, src, dst = pl.kernel(
        _stage_b,
        out_type=[
            jax.ShapeDtypeStruct((NPATHS, 128), f32),
            jax.ShapeDtypeStruct((NPATHS,), jnp.int32),
            jax.ShapeDtypeStruct((NPATHS,), jnp.int32),
        ],
        mesh=mesh,
        scratch_types=[
            pltpu.VMEM((TOTAL + CHK + 8,), jnp.int32),
            pltpu.VMEM((PPW + 16,), jnp.int32),
            pltpu.VMEM((PPW + 16,), jnp.int32),
            pltpu.VMEM((CHK, 128), f32),
            pltpu.VMEM((CHK, 128), f32),
            pltpu.VMEM((PPW, 128), f32),
            pltpu.VMEM((2 * PPW,), jnp.int32),
            pltpu.SemaphoreType.DMA,
            pltpu.SemaphoreType.DMA,
            pltpu.SemaphoreType.DMA,
        ],
        compiler_params=pltpu.CompilerParams(needs_layout_passes=False),
    )(tab, nodes_pad, cs_pad, cs1_pad)

    grid_c = NPATHS // BP
    out_t = pl.pallas_call(
        _stage_c,
        grid=(grid_c,),
        in_specs=[
            pl.BlockSpec((BP, 128), lambda i: (i, 0)),
            pl.BlockSpec(memory_space=pl.ANY),
            pl.BlockSpec((T, FP), lambda i: (0, 0)),
            pl.BlockSpec((T, FP), lambda i: (0, 0)),
        ],
        out_specs=pl.BlockSpec((T, BP), lambda i: (0, i)),
        out_shape=jax.ShapeDtypeStruct((T, NOUT), f32),
        input_output_aliases={1: 0},
    )(sums, big, ic, isn)
    irfs_agg = out_t.T

    diag_idx = jnp.arange(N, dtype=src.dtype)
    coords = jnp.stack([jnp.concatenate([src, diag_idx]),
                        jnp.concatenate([dst, diag_idx])], axis=0)
    return (coords, irfs_agg)


# speedup vs baseline: 1.2988x; 1.0062x over previous
import functools
import math

import jax
import jax.numpy as jnp
from jax import lax
from jax.experimental import pallas as pl
from jax.experimental.pallas import tpu as pltpu
from jax.experimental.pallas import tpu_sc as plsc

N = 50000
TOTAL = 32768
NPATHS = 8192
T = 64
F = 33
FP = 48
BN = 1024
NPAD = 50176
BP = 512
CHK = 128
NW = 32
PPW = NPATHS // NW
HI = jax.lax.Precision.HIGHEST
LO = jax.lax.Precision.DEFAULT


def _softplus(x):
    return jnp.maximum(x, 0.0) + jnp.log1p(jnp.exp(-jnp.abs(x)))


def _stage_a1(p_ref, m_ref, tab_ref):
    p = p_ref[:, :]
    a = _softplus(p[:, 0:1]) + 0.5
    b = _softplus(p[:, 1:2]) + 0.1
    ti = lax.broadcasted_iota(jnp.int32, (1, T), 1).astype(jnp.float32)
    t = ti + 1.0
    u = jnp.exp(a * jnp.log(t) - b * t)
    inv = 1.0 / (jnp.sum(u, axis=-1, keepdims=True) + 1e-12)
    tab_ref[:, :] = jnp.dot(u, m_ref[:, :], precision=LO) * inv


def _stage_a2(p_ref, diag_ref):
    p = p_ref[:, :]
    dn = (((1,), (1,)), ((), ()))
    e0 = jnp.concatenate([jnp.ones((1, 1), jnp.float32),
                          jnp.zeros((1, 1), jnp.float32)], axis=1)
    e1 = jnp.concatenate([jnp.zeros((1, 1), jnp.float32),
                          jnp.ones((1, 1), jnp.float32)], axis=1)
    a_raw = lax.dot_general(e0, p, dn, precision=HI)
    b_raw = lax.dot_general(e1, p, dn, precision=HI)
    a = _softplus(a_raw) + 0.5
    b = _softplus(b_raw) + 0.1
    ti = lax.broadcasted_iota(jnp.int32, (T, 1), 0).astype(jnp.float32)
    tf = jnp.float32(T) - ti
    uf = jnp.exp(a * jnp.log(tf) - b * tf)
    s = jnp.sum(uf, axis=0, keepdims=True)
    diag_ref[:, :] = uf * (1.0 / (s * (1.0 + 1e-8) + 1e-20))


def _stage_b(tab_hbm, nodes_hbm, cs_hbm, cs1_hbm,
             osum_hbm, src_hbm, dst_hbm,
             nodes_v, cs_v, cs1_v, gbuf0_v, gbuf1_v, acc_v, bval_v,
             sem, sem0, sem1):
    wid = lax.axis_index("s") * 2 + lax.axis_index("c")
    p0 = pl.multiple_of(wid * PPW, PPW)
    pltpu.sync_copy(nodes_hbm, nodes_v)
    pltpu.sync_copy(cs_hbm.at[pl.ds(p0, PPW + 16)], cs_v)
    pltpu.sync_copy(cs1_hbm.at[pl.ds(p0, PPW + 16)], cs1_v)

    for k in range(PPW // 16):
        c0 = cs_v[pl.ds(k * 16, 16)]
        c1 = cs1_v[pl.ds(k * 16, 16)]
        s_pos = jnp.clip(c0, 0, TOTAL - 1)
        d_pos = jnp.clip(c1 - 1, 0, TOTAL - 1)
        bval_v[pl.ds(k * 16, 16)] = plsc.load_gather(nodes_v, [s_pos])
        bval_v[pl.ds(PPW + k * 16, 16)] = plsc.load_gather(nodes_v, [d_pos])
    pltpu.sync_copy(bval_v.at[pl.ds(0, PPW)], src_hbm.at[pl.ds(p0, PPW)])
    pltpu.sync_copy(bval_v.at[pl.ds(PPW, PPW)], dst_hbm.at[pl.ds(p0, PPW)])

    one = jnp.ones((16,), jnp.float32)
    zero = jnp.zeros((16,), jnp.float32)

    def zbody(r, _):
        for c in range(FP // 16):
            acc_v[r, pl.ds(c * 16, 16)] = one
        for c in range(FP // 16, 8):
            acc_v[r, pl.ds(c * 16, 16)] = zero
        return 0

    lax.fori_loop(0, PPW, zbody, 0)

    c_lo = cs_v[pl.ds(0, 16)][0]
    c_hi = cs_v[pl.ds(PPW, 16)][0]
    s0 = (c_lo // 8) * 8
    nch = (c_hi - s0 + CHK - 1) // CHK

    gbufs = (gbuf0_v, gbuf1_v)
    sems = (sem0, sem1)

    def _gather(ci, b):
        return pltpu.make_async_copy(
            tab_hbm.at[nodes_v.at[pl.ds(s0 + ci * CHK, CHK)]], gbufs[b],
            sems[b])

    def _advance(rp, i):
        v = cs_v[pl.ds(rp + 1, 16)]
        ind = jnp.minimum(jnp.maximum(i - v + 1, 0), 1)
        return jnp.minimum(rp + jnp.sum(ind), PPW - 1)

    def _flush(st, i):
        rp = st[0]
        for c in range(3):
            acc_v[rp, pl.ds(c * 16, 16)] = st[2 + c]
            acc_v[rp, pl.ds(FP + c * 16, 16)] = st[5 + c]
        rp1 = _advance(rp, i)

        def catchup(r):
            return lax.fori_loop(0, PPW // 16 - 1,
                                 lambda _, rr: _advance(rr, i), r)

        rp2 = lax.cond(rp1 - rp == 16, catchup, lambda r: r, rp1)
        nxt = cs_v[pl.ds(rp2 + 1, 16)][0]
        one = jnp.ones((16,), jnp.float32)
        zero = jnp.zeros((16,), jnp.float32)
        return (rp2, nxt, one, one, one, zero, zero, zero)

    def _make_loop(buf, guarded, s_base):
        def item(j, st):
            i = s_base + j
            fl = st[1] <= i
            if guarded:
                fl = jnp.logical_and(fl, i <= c_hi)
            st = lax.cond(fl, lambda s: _flush(s, i), lambda s: s, st)
            rp, nxt, r0, r1, r2, q0, q1, q2 = st
            b0 = buf[j, pl.ds(0, 16)]
            b1 = buf[j, pl.ds(16, 16)]
            b2 = buf[j, pl.ds(32, 16)]
            d0 = buf[j, pl.ds(FP, 16)]
            d1 = buf[j, pl.ds(FP + 16, 16)]
            d2 = buf[j, pl.ds(FP + 32, 16)]
            n0 = r0 * b0 - q0 * d0
            n1 = r1 * b1 - q1 * d1
            n2 = r2 * b2 - q2 * d2
            m0 = r0 * d0 + q0 * b0
            m1 = r1 * d1 + q1 * b1
            m2 = r2 * d2 + q2 * b2
            if guarded:
                ok = jnp.logical_and(i >= c_lo, i < c_hi)
                n0 = jnp.where(ok, n0, r0)
                n1 = jnp.where(ok, n1, r1)
                n2 = jnp.where(ok, n2, r2)
                m0 = jnp.where(ok, m0, q0)
                m1 = jnp.where(ok, m1, q1)
                m2 = jnp.where(ok, m2, q2)
            return (rp, nxt, n0, n1, n2, m0, m1, m2)

        return item

    def _proc(ci, st, b):
        _gather(ci, b).wait()
        s_base = s0 + ci * CHK
        edge = jnp.logical_or(ci == 0, ci == nch - 1)
        st = lax.cond(
            edge,
            lambda s: lax.fori_loop(0, CHK, _make_loop(gbufs[b], True,
                                                       s_base), s),
            lambda s: lax.fori_loop(0, CHK, _make_loop(gbufs[b], False,
                                                       s_base), s),
            st)

        @pl.when(ci + 2 < nch)
        def _():
            _gather(ci + 2, b).start()

        return st

    @pl.when(0 < nch)
    def _():
        _gather(0, 0).start()

    @pl.when(1 < nch)
    def _():
        _gather(1, 1).start()

    one0 = jnp.ones((16,), jnp.float32)
    zero0 = jnp.zeros((16,), jnp.float32)
    st0 = (jnp.int32(0), cs_v[pl.ds(1, 16)][0],
           one0, one0, one0, zero0, zero0, zero0)

    def chunk_pair(cc, st):
        ci = cc * 2
        st = lax.cond(ci < nch, lambda s: _proc(ci, s, 0), lambda s: s, st)
        st = lax.cond(ci + 1 < nch,
                      lambda s: _proc(ci + 1, s, 1), lambda s: s, st)
        return st

    NCH_MAX = (TOTAL + 7 + CHK - 1) // CHK + 1
    stf = lax.fori_loop(0, (NCH_MAX + 1) // 2, chunk_pair, st0)

    @pl.when(s0 + nch * CHK == c_hi)
    def _():
        rp = stf[0]
        for c in range(3):
            acc_v[rp, pl.ds(c * 16, 16)] = stf[2 + c]
            acc_v[rp, pl.ds(FP + c * 16, 16)] = stf[5 + c]

    pltpu.sync_copy(acc_v, osum_hbm.at[pl.ds(p0, PPW)])


def _stage_c(sum_ref, big_ref, ic_ref, is_ref, o_ref):
    del big_ref
    x = sum_ref[:, :]
    fr = x[:, 0:FP]
    fi = x[:, FP:2 * FP]
    dn = (((1,), (1,)), ((), ()))
    y = (lax.dot_general(ic_ref[:, :], fr, dn, precision=LO)
         + lax.dot_general(is_ref[:, :], fi, dn, precision=LO))
    y = jnp.maximum(y, 0.0)
    o_ref[:, :] = y / (jnp.sum(y, axis=0, keepdims=True) + 1e-8)


@jax.jit
def kernel(params, path_nodes, path_cumsum):
    f32 = jnp.float32
    tt = jnp.arange(T, dtype=f32)
    kk = jnp.arange(FP, dtype=f32)
    kmask = (kk < F).astype(f32)
    ang = 2.0 * math.pi * jnp.outer(tt, kk) / T
    cc = jnp.cos(ang) * kmask
    ss = -jnp.sin(ang) * kmask
    mbasis = jnp.concatenate([cc, ss, jnp.zeros((T, 128 - 2 * FP), f32)],
                             axis=1)
    w = jnp.where((kk == 0) | (kk == F - 1), 1.0, 2.0) * kmask
    tflip = (T - 1) - tt
    ang2 = 2.0 * math.pi * jnp.outer(tflip, kk) / T
    ic = jnp.cos(ang2) * w / T
    isn = -jnp.sin(ang2) * w / T

    grid_a = NPAD // BN
    tab = pl.pallas_call(
        _stage_a1,
        grid=(grid_a,),
        in_specs=[
            pl.BlockSpec((BN, 2), lambda i: (i, 0)),
            pl.BlockSpec((T, 128), lambda i: (0, 0)),
        ],
        out_specs=pl.BlockSpec((BN, 128), lambda i: (i, 0)),
        out_shape=jax.ShapeDtypeStruct((NPAD, 128), f32),
    )(params, mbasis)

    NOUT = NPATHS + N
    big = pl.pallas_call(
        _stage_a2,
        grid=(grid_a,),
        in_specs=[
            pl.BlockSpec((BN, 2), lambda i: (i, 0)),
        ],
        out_specs=pl.BlockSpec((T, BN), lambda i: (0, i + NPATHS // BN)),
        out_shape=jax.ShapeDtypeStruct((T, NOUT), f32),
    )(params)

    nodes_pad = jnp.zeros((TOTAL + CHK + 8,), jnp.int32).at[:TOTAL].set(
        path_nodes.astype(jnp.int32))
    cs_pad = jnp.full((NPATHS + 16,), TOTAL, jnp.int32).at[:NPATHS + 1].set(
        path_cumsum.astype(jnp.int32))
    cs1_pad = jnp.full((NPATHS + 16,), TOTAL, jnp.int32).at[:NPATHS].set(
        path_cumsum[1:].astype(jnp.int32))

    mesh = plsc.VectorSubcoreMesh(core_axis_name="c", subcore_axis_name="s")
    sums, src, dst = pl.kernel(
        _stage_b,
        out_type=[
            jax.ShapeDtypeStruct((NPATHS, 128), f32),
            jax.ShapeDtypeStruct((NPATHS,), jnp.int32),
            jax.ShapeDtypeStruct((NPATHS,), jnp.int32),
        ],
        mesh=mesh,
        scratch_types=[
            pltpu.VMEM((TOTAL + CHK + 8,), jnp.int32),
            pltpu.VMEM((PPW + 16,), jnp.int32),
            pltpu.VMEM((PPW + 16,), jnp.int32),
            pltpu.VMEM((CHK, 128), f32),
            pltpu.VMEM((CHK, 128), f32),
            pltpu.VMEM((PPW, 128), f32),
            pltpu.VMEM((2 * PPW,), jnp.int32),
            pltpu.SemaphoreType.DMA,
            pltpu.SemaphoreType.DMA,
            pltpu.SemaphoreType.DMA,
        ],
        compiler_params=pltpu.CompilerParams(needs_layout_passes=False),
    )(tab, nodes_pad, cs_pad, cs1_pad)

    grid_c = NPATHS // BP
    out_t = pl.pallas_call(
        _stage_c,
        grid=(grid_c,),
        in_specs=[
            pl.BlockSpec((BP, 128), lambda i: (i, 0)),
            pl.BlockSpec(memory_space=pl.ANY),
            pl.BlockSpec((T, FP), lambda i: (0, 0)),
            pl.BlockSpec((T, FP), lambda i: (0, 0)),
        ],
        out_specs=pl.BlockSpec((T, BP), lambda i: (0, i)),
        out_shape=jax.ShapeDtypeStruct((T, NOUT), f32),
        input_output_aliases={1: 0},
    )(sums, big, ic, isn)
    irfs_agg = out_t.T

    diag_idx = jnp.arange(N, dtype=src.dtype)
    coords = jnp.stack([jnp.concatenate([src, diag_idx]),
                        jnp.concatenate([dst, diag_idx])], axis=0)
    return (coords, irfs_agg)


# speedup vs baseline: 1.4851x; 1.1434x over previous
import functools
import math

import jax
import jax.numpy as jnp
from jax import lax
from jax.experimental import pallas as pl
from jax.experimental.pallas import tpu as pltpu
from jax.experimental.pallas import tpu_sc as plsc

N = 50000
TOTAL = 32768
NPATHS = 8192
T = 64
F = 33
FP = 48
BN = 1024
NPAD = 50176
BP = 512
CHK = 128
NW = 32
PPW = NPATHS // NW
HI = jax.lax.Precision.HIGHEST
LO = jax.lax.Precision.DEFAULT


def _softplus(x):
    return jnp.maximum(x, 0.0) + jnp.log1p(jnp.exp(-jnp.abs(x)))


def _stage_a1(p_ref, m_ref, tab_ref):
    p = p_ref[:, :]
    a = _softplus(p[:, 0:1]) + 0.5
    b = _softplus(p[:, 1:2]) + 0.1
    ti = lax.broadcasted_iota(jnp.int32, (1, T), 1).astype(jnp.float32)
    t = ti + 1.0
    u = jnp.exp(a * jnp.log(t) - b * t)
    inv = 1.0 / (jnp.sum(u, axis=-1, keepdims=True) + 1e-12)
    tab_ref[:, :] = jnp.dot(u, m_ref[:, :], precision=LO) * inv


def _stage_a2(p_ref, diag_ref):
    p = p_ref[:, :]
    a = _softplus(p[0:1, :]) + 0.5
    b = _softplus(p[1:2, :]) + 0.1
    ti = lax.broadcasted_iota(jnp.int32, (T, 1), 0).astype(jnp.float32)
    tf = jnp.float32(T) - ti
    uf = jnp.exp(a * jnp.log(tf) - b * tf)
    s = jnp.sum(uf, axis=0, keepdims=True)
    diag_ref[:, :] = uf * (1.0 / (s * (1.0 + 1e-8) + 1e-20))


def _stage_b(tab_hbm, nodes_hbm, cs_hbm, cs1_hbm,
             osum_hbm, src_hbm, dst_hbm,
             nodes_v, cs_v, cs1_v, gbuf0_v, gbuf1_v, acc_v, bval_v,
             sem, sem0, sem1):
    wid = lax.axis_index("s") * 2 + lax.axis_index("c")
    p0 = pl.multiple_of(wid * PPW, PPW)
    pltpu.sync_copy(nodes_hbm, nodes_v)
    pltpu.sync_copy(cs_hbm.at[pl.ds(p0, PPW + 16)], cs_v)
    pltpu.sync_copy(cs1_hbm.at[pl.ds(p0, PPW + 16)], cs1_v)

    for k in range(PPW // 16):
        c0 = cs_v[pl.ds(k * 16, 16)]
        c1 = cs1_v[pl.ds(k * 16, 16)]
        s_pos = jnp.clip(c0, 0, TOTAL - 1)
        d_pos = jnp.clip(c1 - 1, 0, TOTAL - 1)
        bval_v[pl.ds(k * 16, 16)] = plsc.load_gather(nodes_v, [s_pos])
        bval_v[pl.ds(PPW + k * 16, 16)] = plsc.load_gather(nodes_v, [d_pos])
    pltpu.sync_copy(bval_v.at[pl.ds(0, PPW)], src_hbm.at[pl.ds(p0, PPW)])
    pltpu.sync_copy(bval_v.at[pl.ds(PPW, PPW)], dst_hbm.at[pl.ds(p0, PPW)])

    one = jnp.ones((16,), jnp.float32)
    zero = jnp.zeros((16,), jnp.float32)

    def zbody(r, _):
        for c in range(FP // 16):
            acc_v[r, pl.ds(c * 16, 16)] = one
        for c in range(FP // 16, 8):
            acc_v[r, pl.ds(c * 16, 16)] = zero
        return 0

    lax.fori_loop(0, PPW, zbody, 0)

    c_lo = cs_v[pl.ds(0, 16)][0]
    c_hi = cs_v[pl.ds(PPW, 16)][0]
    s0 = (c_lo // 8) * 8
    nch = (c_hi - s0 + CHK - 1) // CHK

    gbufs = (gbuf0_v, gbuf1_v)
    sems = (sem0, sem1)

    def _gather(ci, b):
        return pltpu.make_async_copy(
            tab_hbm.at[nodes_v.at[pl.ds(s0 + ci * CHK, CHK)]], gbufs[b],
            sems[b])

    def _advance(rp, i):
        v = cs_v[pl.ds(rp + 1, 16)]
        ind = jnp.minimum(jnp.maximum(i - v + 1, 0), 1)
        return jnp.minimum(rp + jnp.sum(ind), PPW - 1)

    def _flush(st, i):
        rp = st[0]
        for c in range(3):
            acc_v[rp, pl.ds(c * 16, 16)] = st[2 + c]
            acc_v[rp, pl.ds(FP + c * 16, 16)] = st[5 + c]
        rp1 = _advance(rp, i)

        def catchup(r):
            return lax.fori_loop(0, PPW // 16 - 1,
                                 lambda _, rr: _advance(rr, i), r)

        rp2 = lax.cond(rp1 - rp == 16, catchup, lambda r: r, rp1)
        nxt = cs_v[pl.ds(rp2 + 1, 16)][0]
        one = jnp.ones((16,), jnp.float32)
        zero = jnp.zeros((16,), jnp.float32)
        return (rp2, nxt, one, one, one, zero, zero, zero)

    def _make_loop(buf, guarded, s_base):
        def item(j, st):
            i = s_base + j
            fl = st[1] <= i
            if guarded:
                fl = jnp.logical_and(fl, i <= c_hi)
            st = lax.cond(fl, lambda s: _flush(s, i), lambda s: s, st)
            rp, nxt, r0, r1, r2, q0, q1, q2 = st
            b0 = buf[j, pl.ds(0, 16)]
            b1 = buf[j, pl.ds(16, 16)]
            b2 = buf[j, pl.ds(32, 16)]
            d0 = buf[j, pl.ds(FP, 16)]
            d1 = buf[j, pl.ds(FP + 16, 16)]
            d2 = buf[j, pl.ds(FP + 32, 16)]
            n0 = r0 * b0 - q0 * d0
            n1 = r1 * b1 - q1 * d1
            n2 = r2 * b2 - q2 * d2
            m0 = r0 * d0 + q0 * b0
            m1 = r1 * d1 + q1 * b1
            m2 = r2 * d2 + q2 * b2
            if guarded:
                ok = jnp.logical_and(i >= c_lo, i < c_hi)
                n0 = jnp.where(ok, n0, r0)
                n1 = jnp.where(ok, n1, r1)
                n2 = jnp.where(ok, n2, r2)
                m0 = jnp.where(ok, m0, q0)
                m1 = jnp.where(ok, m1, q1)
                m2 = jnp.where(ok, m2, q2)
            return (rp, nxt, n0, n1, n2, m0, m1, m2)

        return item

    def _proc(ci, st, b):
        _gather(ci, b).wait()
        s_base = s0 + ci * CHK
        edge = jnp.logical_or(ci == 0, ci == nch - 1)
        st = lax.cond(
            edge,
            lambda s: lax.fori_loop(0, CHK, _make_loop(gbufs[b], True,
                                                       s_base), s),
            lambda s: lax.fori_loop(0, CHK, _make_loop(gbufs[b], False,
                                                       s_base), s),
            st)

        @pl.when(ci + 2 < nch)
        def _():
            _gather(ci + 2, b).start()

        return st

    @pl.when(0 < nch)
    def _():
        _gather(0, 0).start()

    @pl.when(1 < nch)
    def _():
        _gather(1, 1).start()

    one0 = jnp.ones((16,), jnp.float32)
    zero0 = jnp.zeros((16,), jnp.float32)
    st0 = (jnp.int32(0), cs_v[pl.ds(1, 16)][0],
           one0, one0, one0, zero0, zero0, zero0)

    def chunk_pair(cc, st):
        ci = cc * 2
        st = lax.cond(ci < nch, lambda s: _proc(ci, s, 0), lambda s: s, st)
        st = lax.cond(ci + 1 < nch,
                      lambda s: _proc(ci + 1, s, 1), lambda s: s, st)
        return st

    NCH_MAX = (TOTAL + 7 + CHK - 1) // CHK + 1
    stf = lax.fori_loop(0, (NCH_MAX + 1) // 2, chunk_pair, st0)

    @pl.when(s0 + nch * CHK == c_hi)
    def _():
        rp = stf[0]
        for c in range(3):
            acc_v[rp, pl.ds(c * 16, 16)] = stf[2 + c]
            acc_v[rp, pl.ds(FP + c * 16, 16)] = stf[5 + c]

    pltpu.sync_copy(acc_v, osum_hbm.at[pl.ds(p0, PPW)])


def _stage_c(sum_ref, big_ref, ic_ref, is_ref, o_ref):
    del big_ref
    x = sum_ref[:, :]
    fr = x[:, 0:FP]
    fi = x[:, FP:2 * FP]
    dn = (((1,), (1,)), ((), ()))
    y = (lax.dot_general(ic_ref[:, :], fr, dn, precision=LO)
         + lax.dot_general(is_ref[:, :], fi, dn, precision=LO))
    y = jnp.maximum(y, 0.0)
    o_ref[:, :] = y / (jnp.sum(y, axis=0, keepdims=True) + 1e-8)


@jax.jit
def kernel(params, path_nodes, path_cumsum):
    f32 = jnp.float32
    tt = jnp.arange(T, dtype=f32)
    kk = jnp.arange(FP, dtype=f32)
    kmask = (kk < F).astype(f32)
    ang = 2.0 * math.pi * jnp.outer(tt, kk) / T
    cc = jnp.cos(ang) * kmask
    ss = -jnp.sin(ang) * kmask
    mbasis = jnp.concatenate([cc, ss, jnp.zeros((T, 128 - 2 * FP), f32)],
                             axis=1)
    params_t = jnp.pad(params.T, ((0, 0), (0, NPAD - N)))
    w = jnp.where((kk == 0) | (kk == F - 1), 1.0, 2.0) * kmask
    tflip = (T - 1) - tt
    ang2 = 2.0 * math.pi * jnp.outer(tflip, kk) / T
    ic = jnp.cos(ang2) * w / T
    isn = -jnp.sin(ang2) * w / T

    grid_a = NPAD // BN
    tab = pl.pallas_call(
        _stage_a1,
        grid=(grid_a,),
        in_specs=[
            pl.BlockSpec((BN, 2), lambda i: (i, 0)),
            pl.BlockSpec((T, 128), lambda i: (0, 0)),
        ],
        out_specs=pl.BlockSpec((BN, 128), lambda i: (i, 0)),
        out_shape=jax.ShapeDtypeStruct((NPAD, 128), f32),
    )(params, mbasis)

    NOUT = NPATHS + N
    big = pl.pallas_call(
        _stage_a2,
        grid=(grid_a,),
        in_specs=[
            pl.BlockSpec((2, BN), lambda i: (0, i)),
        ],
        out_specs=pl.BlockSpec((T, BN), lambda i: (0, i + NPATHS // BN)),
        out_shape=jax.ShapeDtypeStruct((T, NOUT), f32),
    )(params_t)

    nodes_pad = jnp.zeros((TOTAL + CHK + 8,), jnp.int32).at[:TOTAL].set(
        path_nodes.astype(jnp.int32))
    cs_pad = jnp.full((NPATHS + 16,), TOTAL, jnp.int32).at[:NPATHS + 1].set(
        path_cumsum.astype(jnp.int32))
    cs1_pad = jnp.full((NPATHS + 16,), TOTAL, jnp.int32).at[:NPATHS].set(
        path_cumsum[1:].astype(jnp.int32))

    mesh = plsc.VectorSubcoreMesh(core_axis_name="c", subcore_axis_name="s")
    sums, src, dst = pl.kernel(
        _stage_b,
        out_type=[
            jax.ShapeDtypeStruct((NPATHS, 128), f32),
            jax.ShapeDtypeStruct((NPATHS,), jnp.int32),
            jax.ShapeDtypeStruct((NPATHS,), jnp.int32),
        ],
        mesh=mesh,
        scratch_types=[
            pltpu.VMEM((TOTAL + CHK + 8,), jnp.int32),
            pltpu.VMEM((PPW + 16,), jnp.int32),
            pltpu.VMEM((PPW + 16,), jnp.int32),
            pltpu.VMEM((CHK, 128), f32),
            pltpu.VMEM((CHK, 128), f32),
            pltpu.VMEM((PPW, 128), f32),
            pltpu.VMEM((2 * PPW,), jnp.int32),
            pltpu.SemaphoreType.DMA,
            pltpu.SemaphoreType.DMA,
            pltpu.SemaphoreType.DMA,
        ],
        compiler_params=pltpu.CompilerParams(needs_layout_passes=False),
    )(tab, nodes_pad, cs_pad, cs1_pad)

    grid_c = NPATHS // BP
    out_t = pl.pallas_call(
        _stage_c,
        grid=(grid_c,),
        in_specs=[
            pl.BlockSpec((BP, 128), lambda i: (i, 0)),
            pl.BlockSpec(memory_space=pl.ANY),
            pl.BlockSpec((T, FP), lambda i: (0, 0)),
            pl.BlockSpec((T, FP), lambda i: (0, 0)),
        ],
        out_specs=pl.BlockSpec((T, BP), lambda i: (0, i)),
        out_shape=jax.ShapeDtypeStruct((T, NOUT), f32),
        input_output_aliases={1: 0},
    )(sums, big, ic, isn)
    irfs_agg = out_t.T

    diag_idx = jnp.arange(N, dtype=src.dtype)
    coords = jnp.stack([jnp.concatenate([src, diag_idx]),
                        jnp.concatenate([dst, diag_idx])], axis=0)
    return (coords, irfs_agg)
